# Initial kernel scaffold; baseline (speedup 1.0000x reference)
import jax
import jax.numpy as jnp
from jax.experimental import pallas as pl


def kernel(indices, table):
    raise NotImplementedError("write your pallas kernel here")


# speedup vs baseline: 2.9638x; 2.9638x over previous
import functools

import jax
import jax.numpy as jnp
from jax import lax
from jax.experimental import pallas as pl
from jax.experimental.pallas import tpu as pltpu
from jax.experimental.pallas import tpu_sc as plsc

BATCH = 4096
SEQ = 50
D = 128

B_TOTAL = BATCH * SEQ
NUM_WORKERS = 32
B_PER_W = B_TOTAL // NUM_WORKERS
CHUNK = 128
N_CHUNKS = B_PER_W // CHUNK


def _make_gather():
    mesh = plsc.VectorSubcoreMesh(core_axis_name="c", subcore_axis_name="s")

    @functools.partial(
        pl.kernel,
        mesh=mesh,
        out_type=jax.ShapeDtypeStruct((B_TOTAL, D), jnp.float32),
        scratch_types=[
            pltpu.VMEM((B_PER_W,), jnp.int32),
            pltpu.VMEM((CHUNK, D), jnp.float32),
            pltpu.SemaphoreType.DMA,
        ],
    )
    def gather_kernel(idx_hbm, table_hbm, out_hbm, idx_v, rows_v, sem):
        wid = lax.axis_index("s") * 2 + lax.axis_index("c")
        base = wid * B_PER_W
        pltpu.sync_copy(idx_hbm.at[pl.ds(base, B_PER_W)], idx_v)

        def body(j, carry):
            idx_chunk = idx_v.at[pl.ds(j * CHUNK, CHUNK)]
            pltpu.async_copy(table_hbm.at[idx_chunk], rows_v, sem).wait()
            pltpu.sync_copy(rows_v, out_hbm.at[pl.ds(base + j * CHUNK, CHUNK)])
            return carry

        lax.fori_loop(0, N_CHUNKS, body, 0)

    return gather_kernel


_gather = _make_gather()


def kernel(indices, table):
    idx_flat = indices.reshape(B_TOTAL)
    out = _gather(idx_flat, table)
    return out.reshape(BATCH, SEQ, D)


# speedup vs baseline: 3.3033x; 1.1145x over previous
import functools

import jax
import jax.numpy as jnp
from jax import lax
from jax.experimental import pallas as pl
from jax.experimental.pallas import tpu as pltpu
from jax.experimental.pallas import tpu_sc as plsc

BATCH = 4096
SEQ = 50
D = 128

B_TOTAL = BATCH * SEQ
NUM_WORKERS = 32
B_PER_W = B_TOTAL // NUM_WORKERS
CHUNK = 128
N_CHUNKS = B_PER_W // CHUNK
NBUF = 5
assert N_CHUNKS % NBUF == 0


def _make_gather():
    mesh = plsc.VectorSubcoreMesh(core_axis_name="c", subcore_axis_name="s")

    @functools.partial(
        pl.kernel,
        mesh=mesh,
        out_type=jax.ShapeDtypeStruct((B_TOTAL, D), jnp.float32),
        scratch_types=[
            pltpu.VMEM((B_PER_W,), jnp.int32),
            pltpu.VMEM((NBUF, CHUNK, D), jnp.float32),
        ]
        + [pltpu.SemaphoreType.DMA] * (2 * NBUF),
    )
    def gather_kernel(idx_hbm, table_hbm, out_hbm, idx_v, rows_v, *sems):
        gsem = sems[:NBUF]
        ssem = sems[NBUF:]
        wid = lax.axis_index("s") * 2 + lax.axis_index("c")
        base = wid * B_PER_W
        pltpu.sync_copy(idx_hbm.at[pl.ds(base, B_PER_W)], idx_v)

        def gather_desc(j, b):
            idx_chunk = idx_v.at[pl.ds(j * CHUNK, CHUNK)]
            return pltpu.make_async_copy(
                table_hbm.at[idx_chunk], rows_v.at[b], gsem[b])

        def store_desc(j, b):
            return pltpu.make_async_copy(
                rows_v.at[b], out_hbm.at[pl.ds(base + j * CHUNK, CHUNK)],
                ssem[b])

        for b in range(NBUF):
            gather_desc(b, b).start()

        def body(g, carry):
            j0 = g * NBUF
            for b in range(NBUF):
                gather_desc(j0 + b, b).wait()
                store_desc(j0 + b, b).start()
            for b in range(NBUF):
                store_desc(j0 + b, b).wait()
                gather_desc(j0 + NBUF + b, b).start()
            return carry

        lax.fori_loop(0, N_CHUNKS // NBUF - 1, body, 0)

        j0 = N_CHUNKS - NBUF
        for b in range(NBUF):
            gather_desc(j0 + b, b).wait()
            store_desc(j0 + b, b).start()
        for b in range(NBUF):
            store_desc(j0 + b, b).wait()

    return gather_kernel


_gather = _make_gather()


def kernel(indices, table):
    idx_flat = indices.reshape(B_TOTAL)
    out = _gather(idx_flat, table)
    return out.reshape(BATCH, SEQ, D)


# speedup vs baseline: 5.9289x; 1.7948x over previous
import functools

import jax
import jax.numpy as jnp
from jax import lax
from jax.experimental import pallas as pl
from jax.experimental.pallas import tpu as pltpu
from jax.experimental.pallas import tpu_sc as plsc

BATCH = 4096
SEQ = 50
SEQ_PAD = 64
D = 128

NUM_WORKERS = 32
B_PER_W = BATCH // NUM_WORKERS
NBUF = 8
assert B_PER_W % NBUF == 0


def _make_gather():
    mesh = plsc.VectorSubcoreMesh(core_axis_name="c", subcore_axis_name="s")

    @functools.partial(
        pl.kernel,
        mesh=mesh,
        out_type=jax.ShapeDtypeStruct((BATCH, SEQ, D), jnp.float32),
        scratch_types=[
            pltpu.VMEM((B_PER_W * SEQ_PAD,), jnp.int32),
            pltpu.VMEM((NBUF, SEQ, D), jnp.float32),
        ]
        + [pltpu.SemaphoreType.DMA] * (2 * NBUF),
    )
    def gather_kernel(idx_hbm, table_hbm, out_hbm, idx_v, rows_v, *sems):
        gsem = sems[:NBUF]
        ssem = sems[NBUF:]
        wid = lax.axis_index("s") * 2 + lax.axis_index("c")
        bstart = wid * B_PER_W
        pltpu.sync_copy(idx_hbm.at[pl.ds(bstart * SEQ_PAD, B_PER_W * SEQ_PAD)],
                        idx_v)

        def gather_desc(i, b):
            idx_chunk = idx_v.at[pl.ds(i * SEQ_PAD, SEQ)]
            return pltpu.make_async_copy(
                table_hbm.at[idx_chunk], rows_v.at[b], gsem[b])

        def store_desc(i, b):
            return pltpu.make_async_copy(
                rows_v.at[b], out_hbm.at[bstart + i], ssem[b])

        for b in range(NBUF):
            gather_desc(b, b).start()

        def body(g, carry):
            i0 = g * NBUF
            for b in range(NBUF):
                gather_desc(i0 + b, b).wait()
                store_desc(i0 + b, b).start()
            for b in range(NBUF):
                store_desc(i0 + b, b).wait()
                gather_desc(i0 + NBUF + b, b).start()
            return carry

        lax.fori_loop(0, B_PER_W // NBUF - 1, body, 0)

        i0 = B_PER_W - NBUF
        for b in range(NBUF):
            gather_desc(i0 + b, b).wait()
            store_desc(i0 + b, b).start()
        for b in range(NBUF):
            store_desc(i0 + b, b).wait()

    return gather_kernel


_gather = _make_gather()


def kernel(indices, table):
    idx_pad = jnp.pad(indices, ((0, 0), (0, SEQ_PAD - SEQ)))
    out = _gather(idx_pad.reshape(BATCH * SEQ_PAD), table)
    return out


# speedup vs baseline: 10.1801x; 1.7170x over previous
import functools

import jax
import jax.numpy as jnp
from jax import lax
from jax.experimental import pallas as pl
from jax.experimental.pallas import tpu as pltpu
from jax.experimental.pallas import tpu_sc as plsc

BATCH = 4096
SEQ = 50
D = 128

B_TOTAL = BATCH * SEQ
NUM_WORKERS = 32
B_PER_W = B_TOTAL // NUM_WORKERS
CHUNK = 128
N_CHUNKS = B_PER_W // CHUNK
NBUF = 5
assert N_CHUNKS % NBUF == 0


def _make_gather():
    mesh = plsc.VectorSubcoreMesh(core_axis_name="c", subcore_axis_name="s")

    @functools.partial(
        pl.kernel,
        mesh=mesh,
        out_type=jax.ShapeDtypeStruct((B_TOTAL, D), jnp.float32),
        scratch_types=[
            pltpu.VMEM((B_PER_W,), jnp.int32),
            pltpu.VMEM((NBUF, CHUNK, D), jnp.float32),
        ]
        + [pltpu.SemaphoreType.DMA] * (2 * NBUF),
    )
    def gather_kernel(idx_hbm, table_hbm, out_hbm, idx_v, rows_v, *sems):
        gsem = sems[:NBUF]
        ssem = sems[NBUF:]
        wid = lax.axis_index("s") * 2 + lax.axis_index("c")
        base = wid * B_PER_W
        pltpu.sync_copy(idx_hbm.at[pl.ds(base, B_PER_W)], idx_v)

        def gather_desc(j, b):
            idx_chunk = idx_v.at[pl.ds(j * CHUNK, CHUNK)]
            return pltpu.make_async_copy(
                table_hbm.at[idx_chunk], rows_v.at[b], gsem[b])

        def store_desc(j, b):
            return pltpu.make_async_copy(
                rows_v.at[b], out_hbm.at[pl.ds(base + j * CHUNK, CHUNK)],
                ssem[b])

        for b in range(NBUF):
            gather_desc(b, b).start()

        def body(g, carry):
            j0 = g * NBUF
            for b in range(NBUF):
                gather_desc(j0 + b, b).wait()
                store_desc(j0 + b, b).start()
            for b in range(NBUF):
                store_desc(j0 + b, b).wait()
                gather_desc(j0 + NBUF + b, b).start()
            return carry

        lax.fori_loop(0, N_CHUNKS // NBUF - 1, body, 0)

        j0 = N_CHUNKS - NBUF
        for b in range(NBUF):
            gather_desc(j0 + b, b).wait()
            store_desc(j0 + b, b).start()
        for b in range(NBUF):
            store_desc(j0 + b, b).wait()

    return gather_kernel


_gather = _make_gather()


def kernel(indices, table):
    idx_t = indices.T.reshape(B_TOTAL)
    out = _gather(idx_t, table)
    return out.reshape(SEQ, BATCH, D).transpose(1, 0, 2)


# speedup vs baseline: 10.2640x; 1.0082x over previous
import functools

import jax
import jax.numpy as jnp
from jax import lax
from jax.experimental import pallas as pl
from jax.experimental.pallas import tpu as pltpu
from jax.experimental.pallas import tpu_sc as plsc

BATCH = 4096
SEQ = 50
D = 128

B_TOTAL = BATCH * SEQ
NUM_WORKERS = 32
B_PER_W = B_TOTAL // NUM_WORKERS
CHUNK = 80
N_CHUNKS = B_PER_W // CHUNK
NBUF = 10
assert N_CHUNKS % NBUF == 0


def _make_gather():
    mesh = plsc.VectorSubcoreMesh(core_axis_name="c", subcore_axis_name="s")

    @functools.partial(
        pl.kernel,
        mesh=mesh,
        out_type=jax.ShapeDtypeStruct((B_TOTAL, D), jnp.float32),
        scratch_types=[
            pltpu.VMEM((B_PER_W,), jnp.int32),
            pltpu.VMEM((NBUF, CHUNK, D), jnp.float32),
        ]
        + [pltpu.SemaphoreType.DMA] * (2 * NBUF),
    )
    def gather_kernel(idx_hbm, table_hbm, out_hbm, idx_v, rows_v, *sems):
        gsem = sems[:NBUF]
        ssem = sems[NBUF:]
        wid = lax.axis_index("s") * 2 + lax.axis_index("c")
        base = wid * B_PER_W
        pltpu.sync_copy(idx_hbm.at[pl.ds(base, B_PER_W)], idx_v)

        def gather_desc(j, b):
            idx_chunk = idx_v.at[pl.ds(j * CHUNK, CHUNK)]
            return pltpu.make_async_copy(
                table_hbm.at[idx_chunk], rows_v.at[b], gsem[b])

        def store_desc(j, b):
            return pltpu.make_async_copy(
                rows_v.at[b], out_hbm.at[pl.ds(base + j * CHUNK, CHUNK)],
                ssem[b])

        for b in range(NBUF):
            gather_desc(b, b).start()

        def body(g, carry):
            j0 = g * NBUF
            for b in range(NBUF):
                gather_desc(j0 + b, b).wait()
                store_desc(j0 + b, b).start()
            for b in range(NBUF):
                store_desc(j0 + b, b).wait()
                gather_desc(j0 + NBUF + b, b).start()
            return carry

        lax.fori_loop(0, N_CHUNKS // NBUF - 1, body, 0)

        j0 = N_CHUNKS - NBUF
        for b in range(NBUF):
            gather_desc(j0 + b, b).wait()
            store_desc(j0 + b, b).start()
        for b in range(NBUF):
            store_desc(j0 + b, b).wait()

    return gather_kernel


_gather = _make_gather()


def kernel(indices, table):
    idx_t = indices.T.reshape(B_TOTAL)
    out = _gather(idx_t, table)
    return out.reshape(SEQ, BATCH, D).transpose(1, 0, 2)


# speedup vs baseline: 10.3935x; 1.0126x over previous
import functools

import jax
import jax.numpy as jnp
from jax import lax
from jax.experimental import pallas as pl
from jax.experimental.pallas import tpu as pltpu
from jax.experimental.pallas import tpu_sc as plsc

BATCH = 4096
SEQ = 50
D = 128

B_TOTAL = BATCH * SEQ
NUM_WORKERS = 32
CHUNK = 128
N_CHUNKS = SEQ
NBUF = 5
assert N_CHUNKS % NBUF == 0


def _make_gather():
    mesh = plsc.VectorSubcoreMesh(core_axis_name="c", subcore_axis_name="s")

    @functools.partial(
        pl.kernel,
        mesh=mesh,
        out_type=jax.ShapeDtypeStruct((B_TOTAL, D), jnp.float32),
        scratch_types=[
            pltpu.VMEM((N_CHUNKS, CHUNK), jnp.int32),
            pltpu.VMEM((NBUF, CHUNK, D), jnp.float32),
        ]
        + [pltpu.SemaphoreType.DMA] * (2 * NBUF),
    )
    def gather_kernel(idx_hbm, table_hbm, out_hbm, idx_v, rows_v, *sems):
        gsem = sems[:NBUF]
        ssem = sems[NBUF:]
        wid = lax.axis_index("s") * 2 + lax.axis_index("c")
        bcol = wid * CHUNK
        pltpu.sync_copy(idx_hbm.at[:, pl.ds(bcol, CHUNK)], idx_v)

        def gather_desc(j, b):
            return pltpu.make_async_copy(
                table_hbm.at[idx_v.at[j]], rows_v.at[b], gsem[b])

        def store_desc(j, b):
            return pltpu.make_async_copy(
                rows_v.at[b], out_hbm.at[pl.ds(j * BATCH + bcol, CHUNK)],
                ssem[b])

        for b in range(NBUF):
            gather_desc(b, b).start()

        def body(g, carry):
            j0 = g * NBUF
            for b in range(NBUF):
                gather_desc(j0 + b, b).wait()
                store_desc(j0 + b, b).start()
            for b in range(NBUF):
                store_desc(j0 + b, b).wait()
                gather_desc(j0 + NBUF + b, b).start()
            return carry

        lax.fori_loop(0, N_CHUNKS // NBUF - 1, body, 0)

        j0 = N_CHUNKS - NBUF
        for b in range(NBUF):
            gather_desc(j0 + b, b).wait()
            store_desc(j0 + b, b).start()
        for b in range(NBUF):
            store_desc(j0 + b, b).wait()

    return gather_kernel


_gather = _make_gather()


def kernel(indices, table):
    out = _gather(indices.T, table)
    return out.reshape(SEQ, BATCH, D).transpose(1, 0, 2)
